# baseline (speedup 1.0000x reference)
import functools

import jax
import jax.numpy as jnp
from jax import lax
from jax.experimental import pallas as pl
from jax.experimental.pallas import tpu as pltpu
from jax.experimental.pallas import tpu_sc as plsc

NC = 2
NS = 16
L = 16

BATCH = 16384
DIM = 8
NW = NC * NS
BPW = BATCH // NW
CHUNKS = BPW * DIM // L


def _body(idx_hbm, table_hbm, scales_hbm, out_hbm, idx_v, rows_v, scales_v,
          sem):
  wid = lax.axis_index("s") * NC + lax.axis_index("c")
  base = wid * BPW
  pltpu.sync_copy(idx_hbm.at[pl.ds(base, BPW)], idx_v)
  pltpu.sync_copy(scales_hbm, scales_v)
  gather = pltpu.async_copy(table_hbm.at[idx_v], rows_v, sem)
  s = scales_v[...]
  lane = lax.iota(jnp.int32, L)
  row0 = lane >> 3
  col = lane & 7
  gather.wait()

  def mul_step(i, _):
    row = row0 + 2 * i
    v = plsc.load_gather(rows_v, [row, col])
    plsc.store_scatter(rows_v, [row, col], v * s)
    return 0

  lax.fori_loop(0, BPW // 2, mul_step, 0)
  pltpu.sync_copy(rows_v, out_hbm.at[pl.ds(base, BPW)])


@jax.jit
def _run(point_id, cffs_scaled, scales16):
  mesh = plsc.VectorSubcoreMesh(
      core_axis_name="c", subcore_axis_name="s", num_cores=NC,
      num_subcores=NS)
  return pl.kernel(
      _body,
      out_type=jax.ShapeDtypeStruct((BATCH, DIM), jnp.float32),
      mesh=mesh,
      scratch_types=[
          pltpu.VMEM((BPW,), jnp.int32),
          pltpu.VMEM((BPW, DIM), jnp.float32),
          pltpu.VMEM((L,), jnp.float32),
          pltpu.SemaphoreType.DMA,
      ],
      compiler_params=pltpu.CompilerParams(
          needs_layout_passes=False, use_tc_tiling_on_sc=False),
  )(point_id, cffs_scaled, scales16)


def kernel(point_id, cffs_scaled, cff_scales):
  scales16 = jnp.tile(cff_scales, 2)
  return _run(point_id.astype(jnp.int32), cffs_scaled, scales16)
